# baseline (speedup 1.0000x reference)
import functools

import jax
import jax.numpy as jnp
from jax import lax
from jax.experimental import pallas as pl
from jax.experimental.pallas import tpu as pltpu
from jax.experimental.pallas import tpu_sc as plsc

_VOCAB = 1000000
_DIM = 64
_BATCH = 4096
_HIST = 50

_B = _BATCH * _HIST
_NC = 2
_NS = 16
_NW = _NC * _NS
_BPW = _B // _NW
_CHUNK = 128
_NCH = _BPW // _CHUNK

_NT = 7813
_JPW = 244
_NBLK = _JPW // 2
_HALF = _VOCAB // 2

_mesh = plsc.VectorSubcoreMesh(core_axis_name="c", subcore_axis_name="s")


def _diag_tables():
    iota = lax.iota(jnp.int32, 16)
    perms = [(iota + t) & 15 for t in range(16)]
    return iota, perms


@functools.partial(
    pl.kernel,
    out_type=jax.ShapeDtypeStruct((_HALF, 2 * _DIM), jnp.float32),
    mesh=_mesh,
    scratch_types=[
        pltpu.VMEM((_DIM, 2 * _CHUNK), jnp.float32),
        pltpu.VMEM((_DIM, 2 * _CHUNK), jnp.float32),
        pltpu.VMEM((2 * _DIM, _CHUNK), jnp.float32),
        pltpu.VMEM((2 * _DIM, _CHUNK), jnp.float32),
        pltpu.SemaphoreType.DMA,
        pltpu.SemaphoreType.DMA,
        pltpu.SemaphoreType.DMA,
        pltpu.SemaphoreType.DMA,
    ],
    compiler_params=pltpu.CompilerParams(needs_layout_passes=False),
)
def _sc_detile(tT_hbm, out_hbm, bi0, bi1, bo0, bo1, si0, si1, so0, so1):
    w = lax.axis_index("s") * _NC + lax.axis_index("c")
    j0 = w * _JPW
    iota, perms = _diag_tables()
    diotas = [iota + 16 * k for k in range(4)]
    rowoffs = [p >> 1 for p in perms]
    colbases = [((p & 1) << 6) + iota for p in perms]

    def fire_in(m, buf, sem):
        pltpu.async_copy(
            tT_hbm.at[:, pl.ds(128 * (j0 + 2 * m), 2 * _CHUNK)], buf, sem)

    def drain_in(buf, sem):
        pltpu.make_async_copy(
            tT_hbm.at[:, pl.ds(0, 2 * _CHUNK)], buf, sem).wait()

    def transpose(src, dst):
        for c in range(2):
            for k in range(4):
                @plsc.parallel_loop(0, 8)
                def _(vi):
                    v0 = 128 * c + 16 * vi
                    q0 = 64 * c + 8 * vi
                    for t in range(16):
                        g = plsc.load_gather(
                            src, [diotas[k],
                                  jnp.full((16,), v0, jnp.int32) + perms[t]])
                        plsc.store_scatter(
                            dst, [jnp.full((16,), q0, jnp.int32) + rowoffs[t],
                                  colbases[t] + 16 * k], g)

    def fire_out(m, buf, sem):
        pltpu.async_copy(
            buf, out_hbm.at[pl.ds(64 * (j0 + 2 * m), 2 * _DIM)], sem)

    def drain_out(buf, sem):
        pltpu.make_async_copy(buf, out_hbm.at[pl.ds(0, 2 * _DIM)], sem).wait()

    fire_in(0, bi0, si0)
    fire_in(1, bi1, si1)

    def body(i, carry):
        ma = 2 * i
        drain_in(bi0, si0)

        @pl.when(i > 0)
        def _():
            drain_out(bo0, so0)

        transpose(bi0, bo0)
        fire_out(ma, bo0, so0)

        @pl.when(i < (_NBLK // 2 - 1))
        def _():
            fire_in(ma + 2, bi0, si0)

        drain_in(bi1, si1)

        @pl.when(i > 0)
        def _():
            drain_out(bo1, so1)

        transpose(bi1, bo1)
        fire_out(ma + 1, bo1, so1)

        @pl.when(i < (_NBLK // 2 - 1))
        def _():
            fire_in(ma + 3, bi1, si1)

        return carry

    lax.fori_loop(0, _NBLK // 2, body, 0)
    drain_out(bo0, so0)
    drain_out(bo1, so1)

    jr = 7808 + w

    def rem_fire_in():
        pltpu.async_copy(tT_hbm.at[:, pl.ds(128 * jr, _CHUNK)],
                         bi0.at[:, pl.ds(0, _CHUNK)], si0)
        pltpu.make_async_copy(tT_hbm.at[:, pl.ds(0, _CHUNK)],
                              bi0.at[:, pl.ds(0, _CHUNK)], si0).wait()

    def rem_transpose():
        def vbody(vi, carry):
            for k in range(4):
                for t in range(16):
                    g = plsc.load_gather(
                        bi0, [diotas[k],
                              jnp.full((16,), 16 * vi, jnp.int32) + perms[t]])
                    plsc.store_scatter(
                        bo0, [jnp.full((16,), 8 * vi, jnp.int32) + rowoffs[t],
                              colbases[t] + 16 * k], g)
            return carry

        lax.fori_loop(0, 8, vbody, 0)

    @pl.when(w < 4)
    def _():
        rem_fire_in()
        rem_transpose()
        pltpu.sync_copy(bo0.at[pl.ds(0, _DIM)], out_hbm.at[pl.ds(64 * jr, _DIM)])

    @pl.when(w == 4)
    def _():
        rem_fire_in()
        rem_transpose()
        pltpu.sync_copy(bo0.at[pl.ds(0, 32)], out_hbm.at[pl.ds(64 * jr, 32)])


@functools.partial(
    pl.kernel,
    out_type=jax.ShapeDtypeStruct((_B, _DIM), jnp.float32),
    mesh=_mesh,
    scratch_types=[
        pltpu.VMEM((_NCH, _CHUNK), jnp.int32),
        pltpu.VMEM((_NCH, _CHUNK), jnp.int32),
        pltpu.VMEM((_CHUNK, 2 * _DIM), jnp.float32),
        pltpu.VMEM((_CHUNK, 2 * _DIM), jnp.float32),
        pltpu.VMEM((_CHUNK, _DIM), jnp.float32),
        pltpu.SemaphoreType.DMA,
        pltpu.SemaphoreType.DMA,
    ],
    compiler_params=pltpu.CompilerParams(needs_layout_passes=False),
)
def _sc_gather(idx_hbm, par_hbm, table_hbm, out_hbm,
               idx_v, par_v, wa, wb, sel, ga, gb):
    wid = lax.axis_index("s") * _NC + lax.axis_index("c")
    base = wid * _BPW
    iota, perms = _diag_tables()
    diotas = [iota + 16 * k for k in range(4)]
    pltpu.sync_copy(idx_hbm.at[wid], idx_v)
    pltpu.sync_copy(par_hbm.at[wid], par_v)

    def fire(j, buf, sem):
        pltpu.async_copy(table_hbm.at[idx_v.at[j]], buf, sem)

    def drain(buf, sem):
        pltpu.make_async_copy(table_hbm.at[pl.ds(0, _CHUNK)], buf, sem).wait()

    def select(j, wide):
        jvec = jnp.full((16,), j, jnp.int32)

        for k in range(4):
            @plsc.parallel_loop(0, 8)
            def _(r):
                r0 = 16 * r
                for t in range(16):
                    rv = jnp.full((16,), r0, jnp.int32) + perms[t]
                    pv = plsc.load_gather(par_v, [jvec, rv]) << 6
                    g = plsc.load_gather(wide, [rv, pv + diotas[k]])
                    plsc.store_scatter(sel, [rv, diotas[k]], g)

    def flush(j):
        pltpu.sync_copy(sel, out_hbm.at[pl.ds(base + j * _CHUNK, _CHUNK)])

    fire(0, wa, ga)

    def body(i, carry):
        ja = 2 * i
        drain(wa, ga)
        fire(ja + 1, wb, gb)
        select(ja, wa)
        flush(ja)
        drain(wb, gb)

        @pl.when(i < (_NCH // 2 - 1))
        def _():
            fire(ja + 2, wa, ga)

        select(ja + 1, wb)
        flush(ja + 1)
        return carry

    lax.fori_loop(0, _NCH // 2, body, 0)


def kernel(inputs, embeddings):
    table2 = _sc_detile(embeddings.T)
    flat = inputs.astype(jnp.int32).reshape(_B)
    idx = (flat // 2).reshape(_NW, _NCH, _CHUNK)
    par = (flat % 2).reshape(_NW, _NCH, _CHUNK)
    out = _sc_gather(idx, par, table2)
    return out.reshape(_BATCH, _HIST, _DIM)


# speedup vs baseline: 1.1796x; 1.1796x over previous
import functools

import jax
import jax.numpy as jnp
from jax import lax
from jax.experimental import pallas as pl
from jax.experimental.pallas import tpu as pltpu
from jax.experimental.pallas import tpu_sc as plsc

_VOCAB = 1000000
_DIM = 64
_BATCH = 4096
_HIST = 50

_B = _BATCH * _HIST
_NC = 2
_NS = 16
_NW = _NC * _NS
_BPW = _B // _NW
_CHUNK = 128
_NCH = _BPW // _CHUNK
_K = 5
_GROUP = _K * _CHUNK
_NG = _NCH // _K

_mesh = plsc.VectorSubcoreMesh(core_axis_name="c", subcore_axis_name="s")


@functools.partial(
    pl.kernel,
    out_type=jax.ShapeDtypeStruct((_B, _DIM), jnp.float32),
    mesh=_mesh,
    scratch_types=[
        pltpu.VMEM((_NCH, _CHUNK), jnp.int32),
        pltpu.VMEM((_GROUP, _DIM), jnp.float32),
        pltpu.VMEM((_GROUP, _DIM), jnp.float32),
        pltpu.SemaphoreType.DMA,
        pltpu.SemaphoreType.DMA,
        pltpu.SemaphoreType.DMA,
        pltpu.SemaphoreType.DMA,
    ],
    compiler_params=pltpu.CompilerParams(use_tc_tiling_on_sc=False),
)
def _sc_gather(idx_hbm, table_hbm, out_hbm, idx_v, rows_a, rows_b,
               gsem_a, gsem_b, osem_a, osem_b):
    wid = lax.axis_index("s") * _NC + lax.axis_index("c")
    base = wid * _BPW
    pltpu.sync_copy(idx_hbm.at[wid], idx_v)

    def fire_gathers(g, rows, sem):
        for b in range(_K):
            pltpu.async_copy(table_hbm.at[idx_v.at[g * _K + b]],
                             rows.at[pl.ds(b * _CHUNK, _CHUNK)], sem)

    def drain(rows, sem):
        pltpu.make_async_copy(out_hbm.at[pl.ds(0, _GROUP)], rows, sem).wait()

    def fire_out(g, rows, sem):
        pltpu.async_copy(rows, out_hbm.at[pl.ds(base + g * _GROUP, _GROUP)],
                         sem)

    def drain_out(rows, sem):
        pltpu.make_async_copy(rows, out_hbm.at[pl.ds(0, _GROUP)], sem).wait()

    fire_gathers(0, rows_a, gsem_a)
    fire_gathers(1, rows_b, gsem_b)
    drain(rows_a, gsem_a)
    fire_out(0, rows_a, osem_a)

    def body(i, carry):
        g = 1 + 2 * i
        drain(rows_b, gsem_b)
        fire_out(g, rows_b, osem_b)
        drain_out(rows_a, osem_a)
        fire_gathers(g + 1, rows_a, gsem_a)
        drain(rows_a, gsem_a)
        fire_out(g + 1, rows_a, osem_a)
        drain_out(rows_b, osem_b)
        fire_gathers(g + 2, rows_b, gsem_b)
        return carry

    lax.fori_loop(0, _NG // 2 - 1, body, 0)

    drain(rows_b, gsem_b)
    fire_out(_NG - 1, rows_b, osem_b)
    drain_out(rows_a, osem_a)
    drain_out(rows_b, osem_b)


def kernel(inputs, embeddings):
    idx = inputs.astype(jnp.int32).reshape(_NW, _NCH, _CHUNK)
    out = _sc_gather(idx, embeddings)
    return out.reshape(_BATCH, _HIST, _DIM)
